# baseline (speedup 1.0000x reference)
import functools

import numpy as np
import jax
import jax.numpy as jnp
from jax import lax
from jax.experimental import pallas as pl
from jax.experimental.pallas import tpu as pltpu
from jax.experimental.pallas import tpu_sc as plsc

D = 128
V = 100000
B = 1024
L = 200
SCALE = float(np.sqrt(np.float32(D)))

NC, NS = 2, 16
NW = NC * NS
FLAT = B * L
B_PER_W = FLAT // NW
CHUNK = 40
NCH = B_PER_W // CHUNK
NBUF = 4
VPR = D // 16


def _pos_encoding(length, d):
    pos = np.arange(length)[:, np.newaxis]
    i = np.arange(d)[np.newaxis, :]
    angle_rates = 1 / np.power(10000, 2 * (i // 2) / np.float32(d))
    angle_rads = pos * angle_rates
    sines = np.sin(angle_rads[:, 0::2])
    cosines = np.cos(angle_rads[:, 1::2])
    return np.concatenate([sines, cosines], axis=-1).astype(np.float32)


_PE_NP = _pos_encoding(L, D)

_MESH = plsc.VectorSubcoreMesh(core_axis_name="c", subcore_axis_name="s")


@functools.partial(
    pl.kernel,
    out_type=jax.ShapeDtypeStruct((FLAT, D), jnp.float32),
    mesh=_MESH,
    scratch_types=[
        pltpu.VMEM((NCH, CHUNK), jnp.int32),
        pltpu.VMEM((L, D), jnp.float32),
        [pltpu.VMEM((CHUNK, D), jnp.float32) for _ in range(NBUF)],
        [pltpu.SemaphoreType.DMA for _ in range(NBUF)],
        [pltpu.SemaphoreType.DMA for _ in range(NBUF)],
    ],
)
def _sc_embed(seq_hbm, table_hbm, pe_hbm, out_hbm, idx_v, pe_v, bufs, gsems, ssems):
    wid = lax.axis_index("s") * NC + lax.axis_index("c")
    base = wid * B_PER_W
    pltpu.sync_copy(pe_hbm, pe_v)
    pltpu.sync_copy(seq_hbm.at[pl.ds(wid * NCH, NCH), :], idx_v)

    def gather(c, b):
        pltpu.async_copy(table_hbm.at[idx_v.at[c]], bufs[b], gsems[b])

    def gather_drain(b):
        pltpu.make_async_copy(table_hbm.at[idx_v.at[0]], bufs[b], gsems[b]).wait()

    def scatter(c, b):
        pltpu.async_copy(bufs[b], out_hbm.at[pl.ds(base + c * CHUNK, CHUNK)], ssems[b])

    def scatter_drain(b):
        pltpu.make_async_copy(bufs[b], out_hbm.at[pl.ds(base, CHUNK)], ssems[b]).wait()

    gather(0, 0)
    gather(1, 1)

    def outer(g, carry):
        for b in range(NBUF):
            c = g * NBUF + b
            gather_drain(b)
            pe0 = (c % (L // CHUNK)) * CHUNK
            buf = bufs[b]

            def row_body(r, carry2):
                pr = pe0 + r
                for v in range(VPR):
                    sl = pl.ds(v * 16, 16)
                    buf[r, sl] = buf[r, sl] * SCALE + pe_v[pr, sl]
                return carry2

            lax.fori_loop(0, CHUNK, row_body, 0, unroll=2)
            scatter(c, b)

            nb = (b + 2) % NBUF

            @pl.when(c + 2 < NCH)
            def _prefetch():
                @pl.when(c >= 2)
                def _drain_old():
                    scatter_drain(nb)

                gather(c + 2, nb)

        return carry

    lax.fori_loop(0, NCH // NBUF, outer, 0, unroll=False)
    scatter_drain((NCH - 2) % NBUF)
    scatter_drain((NCH - 1) % NBUF)


def kernel(sequence, table):
    seq_flat = sequence.reshape(FLAT // CHUNK, CHUNK).astype(jnp.int32)
    pe = jnp.asarray(_PE_NP)
    out = _sc_embed(seq_flat, table, pe)
    return out.reshape(B, L, D)
